# baseline (speedup 1.0000x reference)
import functools

import numpy as np
import jax
import jax.numpy as jnp
from jax import lax
from jax.experimental import pallas as pl
from jax.experimental.pallas import tpu as pltpu
from jax.experimental.pallas import tpu_sc as plsc

_VOCAB = 1000000
_EMBED = 64
_WINDOW = 2048
_BATCH = 16

_NC = 2
_NS = 16
_L = 16

_BG = _BATCH // 2
_G = _WINDOW // _NS
_SCALE = 8.0


def _pos_encoding_np(length, depth):
    d = depth / 2
    positions = np.arange(length)[:, np.newaxis]
    depths = np.arange(d)[np.newaxis, :] / d
    angle_rates = 1 / 10000 ** depths
    angle_rads = positions * angle_rates
    return np.concatenate(
        [np.sin(angle_rads), np.cos(angle_rads)], axis=-1
    ).astype(np.float32)


def _sc_body(x_hbm, w3_hbm, pe_hbm, out_hbm, idx_v, pe_v, rows_v, out_v, sem):
    cid = lax.axis_index("c")
    sid = lax.axis_index("s")
    wid = sid * _NC + cid
    bg = wid // _NS
    tc = wid % _NS
    t0 = tc * _G

    pltpu.sync_copy(x_hbm.at[pl.ds(bg * _BG, _BG), pl.ds(t0, _G)], idx_v)
    pltpu.sync_copy(pe_hbm.at[pl.ds(t0, _G)], pe_v)

    _H = _G // 4

    for b in range(_BG):

        def half_chunk(h, carry):
            h0 = h * _H

            @plsc.parallel_loop(0, _H // _L, unroll=2)
            def issue_group(g):
                vec = idx_v[b, pl.ds(h0 + g * _L, _L)]
                grp = lax.shift_right_logical(vec, 3)
                dst0 = (
                    (g * _L + lax.iota(jnp.int32, _L)) * 16 + 8 - (vec & 7)
                )
                for j in range(_L):
                    pltpu.async_copy(
                        w3_hbm.at[grp[j]],
                        rows_v.at[pl.ds(dst0[j], 8)],
                        sem,
                    )

            pltpu.make_async_copy(
                out_hbm.at[pl.ds(0, _H * 8)], rows_v.at[pl.ds(0, _H * 8)],
                sem,
            ).wait()

            def fma_group(g, carry2):
                for j in range(_L):
                    s = g * _L + j
                    r = s * 16 + 8
                    for q in range(_EMBED // _L):
                        sl = pl.ds(q * _L, _L)
                        out_v[h0 + s, sl] = (
                            rows_v[r, sl] * _SCALE + pe_v[h0 + s, sl]
                        )
                return carry2

            lax.fori_loop(0, _H // _L, fma_group, 0)
            return carry

        lax.fori_loop(0, 4, half_chunk, 0)

        row0 = (bg * _BG + b) * _WINDOW + t0
        pltpu.sync_copy(out_v, out_hbm.at[pl.ds(row0, _G)])


def kernel(x, W):
    pe = jnp.asarray(_pos_encoding_np(_WINDOW, _EMBED))
    w3 = W.reshape(_VOCAB // 8, 8, _EMBED)
    mesh = plsc.VectorSubcoreMesh(core_axis_name="c", subcore_axis_name="s")
    run = functools.partial(
        pl.kernel,
        mesh=mesh,
        out_type=jax.ShapeDtypeStruct((_BATCH * _WINDOW, _EMBED), jnp.float32),
        scratch_types=[
            pltpu.VMEM((_BG, _G), jnp.int32),
            pltpu.VMEM((_G, _EMBED), jnp.float32),
            pltpu.VMEM((_G * 4, _EMBED), jnp.float32),
            pltpu.VMEM((_G, _EMBED), jnp.float32),
            pltpu.SemaphoreType.DMA,
        ],
    )(_sc_body)
    out = run(x, w3, pe)
    return out.reshape(_BATCH, _WINDOW, _EMBED)


# speedup vs baseline: 1.0756x; 1.0756x over previous
import functools

import numpy as np
import jax
import jax.numpy as jnp
from jax import lax
from jax.experimental import pallas as pl
from jax.experimental.pallas import tpu as pltpu
from jax.experimental.pallas import tpu_sc as plsc

_VOCAB = 1000000
_EMBED = 64
_WINDOW = 2048
_BATCH = 16

_NC = 2
_NS = 16
_L = 16

_BG = _BATCH // 2
_G = _WINDOW // _NS
_SCALE = 8.0


def _pos_encoding_np(length, depth):
    d = depth / 2
    positions = np.arange(length)[:, np.newaxis]
    depths = np.arange(d)[np.newaxis, :] / d
    angle_rates = 1 / 10000 ** depths
    angle_rads = positions * angle_rates
    return np.concatenate(
        [np.sin(angle_rads), np.cos(angle_rads)], axis=-1
    ).astype(np.float32)


def _sc_body(x_hbm, w3_hbm, pe_hbm, out_hbm, idx_v, pe_v, rows_v, out_v,
             sems):
    cid = lax.axis_index("c")
    sid = lax.axis_index("s")
    wid = sid * _NC + cid
    bg = wid // _NS
    tc = wid % _NS
    t0 = tc * _G

    pltpu.sync_copy(x_hbm.at[pl.ds(bg * _BG, _BG), pl.ds(t0, _G)], idx_v)
    pltpu.sync_copy(pe_hbm.at[pl.ds(t0, _G)], pe_v)

    _H = _G // 4

    def b_body(b, carry0):

        def issue(h, buf):
            @plsc.parallel_loop(0, _H // _L, unroll=2)
            def issue_group(g):
                grp = lax.shift_right_logical(
                    idx_v[b, pl.ds(h * _H + g * _L, _L)], 3
                )
                for j in range(_L):
                    pltpu.async_copy(
                        w3_hbm.at[grp[j]],
                        rows_v.at[pl.ds(buf * (_H * 8) + (g * _L + j) * 8,
                                        8)],
                        sems.at[buf],
                    )

        def drain(buf):
            pltpu.make_async_copy(
                out_hbm.at[pl.ds(0, _H * 8)],
                rows_v.at[pl.ds(0, _H * 8)],
                sems.at[buf],
            ).wait()

        def fma(h, buf):
            def fma_group(g, carry):
                sub = idx_v[b, pl.ds(h * _H + g * _L, _L)] & 7
                for j in range(_L):
                    s = g * _L + j
                    r = buf * (_H * 8) + s * 8 + sub[j]
                    for q in range(_EMBED // _L):
                        sl = pl.ds(q * _L, _L)
                        out_v[h * _H + s, sl] = (
                            rows_v[r, sl] * _SCALE + pe_v[h * _H + s, sl]
                        )
                return carry

            lax.fori_loop(0, _H // _L, fma_group, 0)

        issue(0, 0)
        issue(1, 1)
        drain(0)
        fma(0, 0)
        issue(2, 0)
        drain(1)
        fma(1, 1)
        issue(3, 1)
        drain(0)
        fma(2, 0)
        drain(1)
        fma(3, 1)

        row0 = (bg * _BG + b) * _WINDOW + t0
        pltpu.sync_copy(out_v, out_hbm.at[pl.ds(row0, _G)])
        return carry0

    lax.fori_loop(0, _BG, b_body, 0)


def kernel(x, W):
    pe = jnp.asarray(_pos_encoding_np(_WINDOW, _EMBED))
    w3 = W.reshape(_VOCAB // 8, 8, _EMBED)
    mesh = plsc.VectorSubcoreMesh(core_axis_name="c", subcore_axis_name="s")
    run = functools.partial(
        pl.kernel,
        mesh=mesh,
        out_type=jax.ShapeDtypeStruct((_BATCH * _WINDOW, _EMBED), jnp.float32),
        scratch_types=[
            pltpu.VMEM((_BG, _G), jnp.int32),
            pltpu.VMEM((_G, _EMBED), jnp.float32),
            pltpu.VMEM((_G * 4, _EMBED), jnp.float32),
            pltpu.VMEM((_G, _EMBED), jnp.float32),
            pltpu.SemaphoreType.DMA((2,)),
        ],
    )(_sc_body)
    out = run(x, w3, pe)
    return out.reshape(_BATCH, _WINDOW, _EMBED)


# speedup vs baseline: 1.0897x; 1.0131x over previous
import functools

import numpy as np
import jax
import jax.numpy as jnp
from jax import lax
from jax.experimental import pallas as pl
from jax.experimental.pallas import tpu as pltpu
from jax.experimental.pallas import tpu_sc as plsc

_VOCAB = 1000000
_EMBED = 64
_WINDOW = 2048
_BATCH = 16

_NC = 2
_NS = 16
_L = 16

_BG = _BATCH // 2
_G = _WINDOW // _NS
_SCALE = 8.0


def _pos_encoding_np(length, depth):
    d = depth / 2
    positions = np.arange(length)[:, np.newaxis]
    depths = np.arange(d)[np.newaxis, :] / d
    angle_rates = 1 / 10000 ** depths
    angle_rads = positions * angle_rates
    return np.concatenate(
        [np.sin(angle_rads), np.cos(angle_rads)], axis=-1
    ).astype(np.float32)


def _sc_body(x_hbm, w3_hbm, pe_hbm, out_hbm, idx_v, pe_v, rows_v, out_v,
             sems, out_sem):
    cid = lax.axis_index("c")
    sid = lax.axis_index("s")
    wid = sid * _NC + cid
    bg = wid // _NS
    tc = wid % _NS
    t0 = tc * _G

    pltpu.sync_copy(x_hbm.at[pl.ds(bg * _BG, _BG), pl.ds(t0, _G)], idx_v)
    pltpu.sync_copy(pe_hbm.at[pl.ds(t0, _G)], pe_v)

    _H = _G // 4

    def b_body(b, carry0):

        def issue(h, buf):
            @plsc.parallel_loop(0, _H // _L, unroll=2)
            def issue_group(g):
                grp = lax.shift_right_logical(
                    idx_v[b, pl.ds(h * _H + g * _L, _L)], 3
                )
                for j in range(_L):
                    pltpu.async_copy(
                        w3_hbm.at[grp[j]],
                        rows_v.at[pl.ds(buf * (_H * 8) + (g * _L + j) * 8,
                                        8)],
                        sems.at[buf],
                    )

        def drain(buf):
            pltpu.make_async_copy(
                out_hbm.at[pl.ds(0, _H * 8)],
                rows_v.at[pl.ds(0, _H * 8)],
                sems.at[buf],
            ).wait()

        def fma(h, buf):
            def fma_group(g, carry):
                sub = idx_v[b, pl.ds(h * _H + g * _L, _L)] & 7
                for j in range(_L):
                    s = g * _L + j
                    r = buf * (_H * 8) + s * 8 + sub[j]
                    for q in range(_EMBED // _L):
                        sl = pl.ds(q * _L, _L)
                        out_v[h * _H + s, sl] = (
                            rows_v[r, sl] * _SCALE + pe_v[h * _H + s, sl]
                        )
                return carry

            lax.fori_loop(0, _H // _L, fma_group, 0)

        issue(0, 0)
        issue(1, 1)
        drain(0)
        @pl.when(b > 0)
        def _():
            pltpu.make_async_copy(
                out_hbm.at[pl.ds(0, _G)], out_v, out_sem
            ).wait()

        fma(0, 0)
        issue(2, 0)
        drain(1)
        fma(1, 1)
        issue(3, 1)
        drain(0)
        fma(2, 0)
        drain(1)
        fma(3, 1)

        row0 = (bg * _BG + b) * _WINDOW + t0
        pltpu.async_copy(out_v, out_hbm.at[pl.ds(row0, _G)], out_sem)
        return carry0

    lax.fori_loop(0, _BG, b_body, 0)
    pltpu.make_async_copy(out_hbm.at[pl.ds(0, _G)], out_v, out_sem).wait()


def kernel(x, W):
    pe = jnp.asarray(_pos_encoding_np(_WINDOW, _EMBED))
    w3 = W.reshape(_VOCAB // 8, 8, _EMBED)
    mesh = plsc.VectorSubcoreMesh(core_axis_name="c", subcore_axis_name="s")
    run = functools.partial(
        pl.kernel,
        mesh=mesh,
        out_type=jax.ShapeDtypeStruct((_BATCH * _WINDOW, _EMBED), jnp.float32),
        scratch_types=[
            pltpu.VMEM((_BG, _G), jnp.int32),
            pltpu.VMEM((_G, _EMBED), jnp.float32),
            pltpu.VMEM((_G * 4, _EMBED), jnp.float32),
            pltpu.VMEM((_G, _EMBED), jnp.float32),
            pltpu.SemaphoreType.DMA((2,)),
            pltpu.SemaphoreType.DMA,
        ],
    )(_sc_body)
    out = run(x, w3, pe)
    return out.reshape(_BATCH, _WINDOW, _EMBED)
